# baseline (speedup 1.0000x reference)
import functools

import jax
import jax.numpy as jnp
import numpy as np
from jax.experimental import pallas as pl

L_MAX = 3
Q = 32
PAIR = Q * Q
G = PAIR // 128


def _expansion_consts():
    rep = np.zeros((Q, PAIR), dtype=np.float32)
    tile = np.zeros((Q, 128), dtype=np.float32)
    for q in range(Q):
        rep[q, q * Q:(q + 1) * Q] = 1.0
        tile[q, q::Q] = 1.0
    return rep.astype(jnp.bfloat16), tile.astype(jnp.bfloat16)


_REP, _TILE = _expansion_consts()


def _ps_kernel(a0, a1, a2, a3, b0, b1, b2, b3, rep, tile, out_ref):
    a_refs = (a0, a1, a2, a3)
    b_refs = (b0, b1, b2, b3)
    for l in range(L_MAX + 1):
        ml = 2 * l + 1
        cg = np.float32(1.0 / np.sqrt(ml))
        a = a_refs[l][...].astype(jnp.bfloat16)
        b = b_refs[l][...].astype(jnp.bfloat16)
        rows = [None] * G
        for m in range(ml):
            am = a[:, m * Q:(m + 1) * Q]
            bm = b[:, m * Q:(m + 1) * Q]
            ar = jnp.dot(am, rep[...], preferred_element_type=jnp.float32)
            bt = jnp.dot(bm, tile[...], preferred_element_type=jnp.float32)
            for g in range(G):
                term = ar[:, g * 128:(g + 1) * 128] * bt
                rows[g] = term if rows[g] is None else rows[g] + term
        out_ref[:, l * PAIR:(l + 1) * PAIR] = jnp.concatenate(rows, axis=1) * cg


@functools.partial(jax.jit, static_argnames=())
def kernel(density_nu_l0, density_nu_l1, density_nu_l2, density_nu_l3,
           density_1_l0, density_1_l1, density_1_l2, density_1_l3):
    n = density_nu_l0.shape[0]
    sb = 128
    grid = (n // sb,)

    nus = (density_nu_l0, density_nu_l1, density_nu_l2, density_nu_l3)
    d1s = (density_1_l0, density_1_l1, density_1_l2, density_1_l3)
    nus = tuple(x.reshape(n, -1).astype(jnp.bfloat16 if l else jnp.float32)
                for l, x in enumerate(nus))
    d1s = tuple(x.reshape(n, -1).astype(jnp.bfloat16 if l else jnp.float32)
                for l, x in enumerate(d1s))

    in_specs = []
    for l in range(L_MAX + 1):
        in_specs.append(pl.BlockSpec((sb, (2 * l + 1) * Q), lambda i: (i, 0)))
    in_specs = in_specs + in_specs
    in_specs.append(pl.BlockSpec(_REP.shape, lambda i: (0, 0)))
    in_specs.append(pl.BlockSpec(_TILE.shape, lambda i: (0, 0)))

    out = pl.pallas_call(
        _ps_kernel,
        grid=grid,
        in_specs=in_specs,
        out_specs=pl.BlockSpec((sb, (L_MAX + 1) * PAIR), lambda i: (i, 0)),
        out_shape=jax.ShapeDtypeStruct((n, (L_MAX + 1) * PAIR), jnp.float32),
    )(*nus, *d1s, _REP, _TILE)
    return out


# speedup vs baseline: 1.0283x; 1.0283x over previous
import functools

import jax
import jax.numpy as jnp
import numpy as np
from jax.experimental import pallas as pl

L_MAX = 3
Q = 32
PAIR = Q * Q
G = PAIR // 128


def _expansion_consts():
    rep = np.zeros((Q, PAIR), dtype=np.float32)
    tile = np.zeros((Q, 128), dtype=np.float32)
    for q in range(Q):
        rep[q, q * Q:(q + 1) * Q] = 1.0
        tile[q, q::Q] = 1.0
    return rep.astype(jnp.bfloat16), tile.astype(jnp.bfloat16)


_REP, _TILE = _expansion_consts()


def _ps_kernel(a0, a1, a2, a3, b0, b1, b2, b3, rep, tile, out_ref):
    a_refs = (a0, a1, a2, a3)
    b_refs = (b0, b1, b2, b3)
    for l in range(L_MAX + 1):
        ml = 2 * l + 1
        cg = np.float32(1.0 / np.sqrt(ml))
        a = a_refs[l][...]
        b = b_refs[l][...]
        rows = [None] * G
        for m in range(ml):
            am = a[:, m * Q:(m + 1) * Q]
            bm = b[:, m * Q:(m + 1) * Q]
            ar = jnp.dot(am, rep[...], preferred_element_type=jnp.float32)
            bt = jnp.dot(bm, tile[...], preferred_element_type=jnp.float32)
            for g in range(G):
                term = ar[:, g * 128:(g + 1) * 128] * bt
                rows[g] = term if rows[g] is None else rows[g] + term
        out_ref[:, l * PAIR:(l + 1) * PAIR] = jnp.concatenate(rows, axis=1) * cg


@functools.partial(jax.jit, static_argnames=())
def kernel(density_nu_l0, density_nu_l1, density_nu_l2, density_nu_l3,
           density_1_l0, density_1_l1, density_1_l2, density_1_l3):
    n = density_nu_l0.shape[0]
    sb = 128
    grid = (n // sb,)

    nus = (density_nu_l0, density_nu_l1, density_nu_l2, density_nu_l3)
    d1s = (density_1_l0, density_1_l1, density_1_l2, density_1_l3)
    nus = tuple(x.reshape(n, -1).astype(jnp.bfloat16) for x in nus)
    d1s = tuple(x.reshape(n, -1).astype(jnp.bfloat16) for x in d1s)

    in_specs = []
    for l in range(L_MAX + 1):
        in_specs.append(pl.BlockSpec((sb, (2 * l + 1) * Q), lambda i: (i, 0)))
    in_specs = in_specs + in_specs
    in_specs.append(pl.BlockSpec(_REP.shape, lambda i: (0, 0)))
    in_specs.append(pl.BlockSpec(_TILE.shape, lambda i: (0, 0)))

    out = pl.pallas_call(
        _ps_kernel,
        grid=grid,
        in_specs=in_specs,
        out_specs=pl.BlockSpec((sb, (L_MAX + 1) * PAIR), lambda i: (i, 0)),
        out_shape=jax.ShapeDtypeStruct((n, (L_MAX + 1) * PAIR), jnp.float32),
    )(*nus, *d1s, _REP, _TILE)
    return out
